# baseline (speedup 1.0000x reference)
import functools

import jax
import jax.numpy as jnp
from jax import lax
from jax.experimental import pallas as pl
from jax.experimental.pallas import tpu as pltpu
from jax.experimental.pallas import tpu_sc as plsc

H, W, A = 100, 152, 3
N_ANCHORS = H * W * A
PRE_NMS_TOPK = 2000
POST_NMS_TOPK = 1000
NMS_THRESH = 0.7
IMG_H, IMG_W = 800.0, 1216.0

M_PAD = 2048
CHUNK = 128
N_CHUNKS = M_PAD // CHUNK

R_SEL = 384
N_SEL = R_SEL * 128
ROWS_PER_TILE = R_SEL // 32
ITEMS_PER_TILE = ROWS_PER_TILE * 128
OUT_ROWS = M_PAD + 128
PAYC = 128
STAGE = 512

HP, WP = H + 2, W + 2
P_VALID = HP * WP
T_CONV = 512
N_T = (P_VALID + T_CONV - 1) // T_CONV
P_PAD = N_T * T_CONV
MARGIN = WP + 1
X_ROWS = ((P_PAD + 2 * MARGIN + T_CONV - 1) // T_CONV) * T_CONV
WIDE = ((T_CONV + 2 * MARGIN + 511) // 512) * 512
_OFFS = tuple((dh - 1) * WP + (dw - 1) + MARGIN
              for dh in range(3) for dw in range(3))

_INTERPRET = False


def _conv_body(x_ref, w_ref, wh_ref, bc_ref, bh_ref, o_ref):
    t = pl.program_id(0)
    xw = x_ref[pl.ds(t * T_CONV, WIDE), :]
    acc = jnp.zeros((T_CONV, 256), jnp.float32)
    for k in range(9):
        off = _OFFS[k]
        acc += jnp.dot(xw[off:off + T_CONV, :], w_ref[k],
                       preferred_element_type=jnp.float32)
    xr = jax.nn.relu(acc + bc_ref[...])
    o_ref[...] = jnp.dot(xr, wh_ref[...],
                         preferred_element_type=jnp.float32) + bh_ref[...]


def _conv_head(features, conv_w, conv_b, obj_w, obj_b, delta_w, delta_b):
    xp = jnp.pad(features[0], ((0, 0), (1, 1), (1, 1)))
    xp = xp.reshape(256, P_VALID).T
    xb = jnp.zeros((X_ROWS, 256), jnp.float32)
    xb = jax.lax.dynamic_update_slice(xb, xp, (MARGIN, 0))
    w9 = jnp.transpose(conv_w, (2, 3, 1, 0)).reshape(9, 256, 256)
    wh = jnp.concatenate([obj_w[:, :, 0, 0], delta_w[:, :, 0, 0]], axis=0).T
    bh = jnp.concatenate([obj_b, delta_b])[None, :]
    out = pl.pallas_call(
        _conv_body,
        grid=(N_T,),
        in_specs=[
            pl.BlockSpec((X_ROWS, 256), lambda t: (0, 0)),
            pl.BlockSpec((9, 256, 256), lambda t: (0, 0, 0)),
            pl.BlockSpec((256, 15), lambda t: (0, 0)),
            pl.BlockSpec((1, 256), lambda t: (0, 0)),
            pl.BlockSpec((1, 15), lambda t: (0, 0)),
        ],
        out_specs=pl.BlockSpec((T_CONV, 15), lambda t: (t, 0)),
        out_shape=jax.ShapeDtypeStruct((P_PAD, 15), jnp.float32),
        interpret=_INTERPRET,
    )(xb, w9, wh, conv_b[None, :], bh)
    hw = out[:P_VALID].reshape(HP, WP, 15)[1:1 + H, 1:1 + W]
    scores = hw[..., :A].reshape(N_ANCHORS)
    deltas = hw[..., A:].reshape(H, W, A, 4).reshape(N_ANCHORS, 4)
    return scores, deltas


def _excl_cumsum(x, u128, l384, ones128):
    within = jnp.dot(x, u128, preferred_element_type=jnp.float32)
    rowsum = jnp.dot(x, ones128, preferred_element_type=jnp.float32)
    base = jnp.dot(l384, rowsum, preferred_element_type=jnp.float32)
    return base + within


def _select_body(s_ref, tid_ref, sidx_ref):
    s = s_ref[...]
    b = jax.lax.bitcast_convert_type(s, jnp.int32)
    key = b ^ jnp.right_shift(b, 31) & jnp.int32(0x7FFFFFFF)
    kf = jnp.float32(PRE_NMS_TOPK)

    nn0 = jnp.sum((key >= 0).astype(jnp.float32))
    p0 = nn0 >= kf
    lo0 = jnp.where(p0, jnp.int32(0), jnp.int32(-2147483648))
    hi0 = jnp.where(p0, jnp.int32(2147483647), jnp.int32(0))

    def bs(_, carry):
        lo, hi = carry
        mid = lo + jnp.right_shift(hi - lo, 1)
        cnt = jnp.sum((key >= mid).astype(jnp.float32))
        p = cnt >= kf
        return jnp.where(p, mid, lo), jnp.where(p, hi, mid)

    t_key, _ = jax.lax.fori_loop(0, 31, bs, (lo0, hi0))

    gt = (key > t_key)
    need = kf - jnp.sum(gt.astype(jnp.float32))
    eq = (key == t_key)

    u128 = (jax.lax.broadcasted_iota(jnp.int32, (CHUNK, CHUNK), 0)
            < jax.lax.broadcasted_iota(jnp.int32, (CHUNK, CHUNK), 1)
            ).astype(jnp.float32)
    l384 = (jax.lax.broadcasted_iota(jnp.int32, (R_SEL, R_SEL), 1)
            < jax.lax.broadcasted_iota(jnp.int32, (R_SEL, R_SEL), 0)
            ).astype(jnp.float32)
    ones128 = jnp.ones((CHUNK, 1), jnp.float32)

    eqex = _excl_cumsum(eq.astype(jnp.float32), u128, l384, ones128)
    sel = gt | (eq & (eqex < need))
    slot = _excl_cumsum(sel.astype(jnp.float32), u128, l384, ones128)
    sidx_ref[...] = jnp.where(sel, slot.astype(jnp.int32),
                              M_PAD + tid_ref[...])


def _select_slots(s_pad, tid):
    return pl.pallas_call(
        _select_body,
        out_shape=jax.ShapeDtypeStruct((R_SEL, 128), jnp.int32),
        interpret=_INTERPRET,
    )(s_pad, tid)


def _sc_scatter(payload, sidx):
    mesh = plsc.VectorSubcoreMesh(core_axis_name="c", subcore_axis_name="s")

    @functools.partial(
        pl.kernel, mesh=mesh,
        out_type=jax.ShapeDtypeStruct((OUT_ROWS, PAYC), jnp.float32),
        scratch_types=[
            pltpu.VMEM((ROWS_PER_TILE, 128), jnp.int32),
            pltpu.VMEM((STAGE, PAYC), jnp.float32),
            pltpu.SemaphoreType.DMA,
        ],
    )
    def scatter(pay_hbm, sidx_hbm, out_hbm, idx_v, pay_v, sem):
        wid = lax.axis_index("s") * 2 + lax.axis_index("c")
        pltpu.sync_copy(sidx_hbm.at[wid], idx_v)
        for g in range(ITEMS_PER_TILE // STAGE):
            pltpu.sync_copy(
                pay_hbm.at[pl.ds(wid * ITEMS_PER_TILE + g * STAGE, STAGE)],
                pay_v)
            cps = [pltpu.async_copy(
                pay_v.at[pl.ds(j * 128, 128)],
                out_hbm.at[idx_v.at[g * (STAGE // 128) + j]], sem)
                for j in range(STAGE // 128)]
            for cp in cps:
                cp.wait()

    return scatter(payload, sidx.reshape(32, ROWS_PER_TILE, 128))


def _decode8(a0, a1, a2, a3, d0, d1, d2, d3):
    w = a2 - a0
    h = a3 - a1
    cx = a0 + 0.5 * w
    cy = a1 + 0.5 * h
    dw = jnp.clip(d2, -4.0, 4.0)
    dh = jnp.clip(d3, -4.0, 4.0)
    pcx = d0 * w + cx
    pcy = d1 * h + cy
    pw = jnp.exp(dw) * w
    ph = jnp.exp(dh) * h
    x1 = jnp.clip(pcx - 0.5 * pw, 0.0, IMG_W)
    y1 = jnp.clip(pcy - 0.5 * ph, 0.0, IMG_H)
    x2 = jnp.clip(pcx + 0.5 * pw, 0.0, IMG_W)
    y2 = jnp.clip(pcy + 0.5 * ph, 0.0, IMG_H)
    return x1, y1, x2, y2


def _nms_body(pay_ref, payt_ref, out_ref, q_ref, c_ref, bx_ref):
    valid = (jax.lax.broadcasted_iota(jnp.int32, (1, M_PAD), 1)
             < PRE_NMS_TOPK)
    vf = valid.astype(jnp.float32)
    rows = [payt_ref[i:i + 1, :] * vf for i in range(10)]
    s_r, i_r = rows[0], rows[9]
    x1r, y1r, x2r, y2r = _decode8(*rows[1:9])
    area_r = (x2r - x1r) * (y2r - y1r)

    for c in range(N_CHUNKS):
        p = pay_ref[c * CHUNK:(c + 1) * CHUNK, :]
        if (c + 1) * CHUNK > PRE_NMS_TOPK:
            rmask = (c * CHUNK + jax.lax.broadcasted_iota(
                jnp.int32, (CHUNK, 1), 0)) < PRE_NMS_TOPK
            p = p * rmask.astype(jnp.float32)
        s_i, i_i = p[:, 0:1], p[:, 9:10]
        x1i, y1i, x2i, y2i = _decode8(*(p[:, i:i + 1] for i in range(1, 9)))
        bx_ref[c * CHUNK:(c + 1) * CHUNK, :] = jnp.concatenate(
            [x1i, y1i, x2i, y2i], axis=1)
        area_i = (x2i - x1i) * (y2i - y1i)
        before = (s_i > s_r) | ((s_i == s_r) & (i_i < i_r))
        c_ref[c * CHUNK:(c + 1) * CHUNK, :] = before.astype(jnp.bfloat16)
        wx = jnp.clip(jnp.minimum(x2i, x2r) - jnp.maximum(x1i, x1r), 0.0)
        wy = jnp.clip(jnp.minimum(y2i, y2r) - jnp.maximum(y1i, y1r), 0.0)
        inter = wx * wy
        iou = inter / (area_i + area_r - inter + 1e-9)
        q_ref[c * CHUNK:(c + 1) * CHUNK, :] = (
            (iou > NMS_THRESH) & before).astype(jnp.bfloat16)

    k0 = vf

    def cond(carry):
        return carry[1]

    def body(carry):
        k, _ = carry
        cnt = jnp.dot(k.astype(jnp.bfloat16), q_ref[...],
                      preferred_element_type=jnp.float32)
        k_new = jnp.where((cnt == 0.0) & valid, 1.0, 0.0)
        return k_new, jnp.any(k_new != k)

    k, _ = jax.lax.while_loop(cond, body, (k0, jnp.bool_(True)))

    slot = jnp.dot(k.astype(jnp.bfloat16), c_ref[...],
                   preferred_element_type=jnp.float32)
    rr = jax.lax.broadcasted_iota(jnp.int32, (POST_NMS_TOPK, CHUNK), 0)
    acc = jnp.zeros((POST_NMS_TOPK, 4), jnp.float32)
    for c in range(N_CHUNKS):
        kc = k[:, c * CHUNK:(c + 1) * CHUNK]
        sc = slot[:, c * CHUNK:(c + 1) * CHUNK]
        pt = ((sc.astype(jnp.int32) == rr) & (kc == 1.0)).astype(jnp.float32)
        bc = bx_ref[c * CHUNK:(c + 1) * CHUNK, :]
        acc += jnp.dot(pt, bc, preferred_element_type=jnp.float32)
    out_ref[...] = acc


def _nms_compact(compacted):
    pay = compacted[:M_PAD, :16]
    payt = pay.T
    return pl.pallas_call(
        _nms_body,
        out_shape=jax.ShapeDtypeStruct((POST_NMS_TOPK, 4), jnp.float32),
        scratch_shapes=[pltpu.VMEM((M_PAD, M_PAD), jnp.bfloat16),
                        pltpu.VMEM((M_PAD, M_PAD), jnp.bfloat16),
                        pltpu.VMEM((M_PAD, 4), jnp.float32)],
        interpret=_INTERPRET,
    )(pay, payt)


def kernel(features, conv_w, conv_b, obj_w, obj_b, delta_w, delta_b, anchors):
    scores, deltas = _conv_head(
        features, conv_w, conv_b, obj_w, obj_b, delta_w, delta_b)
    npad = N_SEL - N_ANCHORS
    s_pad = jnp.concatenate(
        [scores, jnp.full((npad,), -jnp.inf, jnp.float32)])
    idxf = jnp.arange(N_SEL, dtype=jnp.float32)[:, None]
    payload = jnp.concatenate([
        s_pad[:, None],
        jnp.concatenate([anchors, jnp.zeros((npad, 4), jnp.float32)], axis=0),
        jnp.concatenate([deltas, jnp.zeros((npad, 4), jnp.float32)], axis=0),
        idxf,
        jnp.zeros((N_SEL, PAYC - 10), jnp.float32),
    ], axis=1)
    tid = (jnp.arange(R_SEL, dtype=jnp.int32) // ROWS_PER_TILE)[:, None]
    sidx = _select_slots(s_pad.reshape(R_SEL, 128), tid)
    compacted = _sc_scatter(payload, sidx)
    return _nms_compact(compacted)


# speedup vs baseline: 1.7141x; 1.7141x over previous
import functools

import jax
import jax.numpy as jnp
from jax import lax
from jax.experimental import pallas as pl
from jax.experimental.pallas import tpu as pltpu
from jax.experimental.pallas import tpu_sc as plsc

H, W, A = 100, 152, 3
N_ANCHORS = H * W * A
PRE_NMS_TOPK = 2000
POST_NMS_TOPK = 1000
NMS_THRESH = 0.7
IMG_H, IMG_W = 800.0, 1216.0

M_PAD = 2048
CHUNK = 128
N_CHUNKS = M_PAD // CHUNK
GB = M_PAD // 32
ADC = 128

HP, WP = H + 2, W + 2
P_VALID = HP * WP
T_CONV = 512
N_T = (P_VALID + T_CONV - 1) // T_CONV
P_PAD = N_T * T_CONV
MARGIN = WP + 1
X_ROWS = ((P_PAD + 2 * MARGIN + T_CONV - 1) // T_CONV) * T_CONV
WIDE = ((T_CONV + 2 * MARGIN + 511) // 512) * 512
_OFFS = tuple((dh - 1) * WP + (dw - 1) + MARGIN
              for dh in range(3) for dw in range(3))

_INTERPRET = False


def _conv_body(x_ref, w_ref, wh_ref, bc_ref, bh_ref, o_ref):
    t = pl.program_id(0)
    xw = x_ref[pl.ds(t * T_CONV, WIDE), :]
    acc = jnp.zeros((T_CONV, 256), jnp.float32)
    for k in range(9):
        off = _OFFS[k]
        acc += jnp.dot(xw[off:off + T_CONV, :], w_ref[k],
                       preferred_element_type=jnp.float32)
    xr = jax.nn.relu(acc + bc_ref[...])
    o_ref[...] = jnp.dot(xr, wh_ref[...],
                         preferred_element_type=jnp.float32) + bh_ref[...]


def _conv_head(features, conv_w, conv_b, obj_w, obj_b, delta_w, delta_b):
    xp = jnp.pad(features[0], ((0, 0), (1, 1), (1, 1)))
    xp = xp.reshape(256, P_VALID).T
    xb = jnp.zeros((X_ROWS, 256), jnp.float32)
    xb = jax.lax.dynamic_update_slice(xb, xp, (MARGIN, 0))
    w9 = jnp.transpose(conv_w, (2, 3, 1, 0)).reshape(9, 256, 256)
    wh = jnp.concatenate([obj_w[:, :, 0, 0], delta_w[:, :, 0, 0]], axis=0).T
    bh = jnp.concatenate([obj_b, delta_b])[None, :]
    out = pl.pallas_call(
        _conv_body,
        grid=(N_T,),
        in_specs=[
            pl.BlockSpec((X_ROWS, 256), lambda t: (0, 0)),
            pl.BlockSpec((9, 256, 256), lambda t: (0, 0, 0)),
            pl.BlockSpec((256, 15), lambda t: (0, 0)),
            pl.BlockSpec((1, 256), lambda t: (0, 0)),
            pl.BlockSpec((1, 15), lambda t: (0, 0)),
        ],
        out_specs=pl.BlockSpec((T_CONV, 15), lambda t: (t, 0)),
        out_shape=jax.ShapeDtypeStruct((P_PAD, 15), jnp.float32),
        interpret=_INTERPRET,
    )(xb, w9, wh, conv_b[None, :], bh)
    hw = out[:P_VALID].reshape(HP, WP, 15)[1:1 + H, 1:1 + W]
    scores = hw[..., :A].reshape(N_ANCHORS)
    deltas = hw[..., A:].reshape(H, W, A, 4).reshape(N_ANCHORS, 4)
    return scores, deltas


def _sc_gather(ad_table, top_idx):
    mesh = plsc.VectorSubcoreMesh(core_axis_name="c", subcore_axis_name="s")

    @functools.partial(
        pl.kernel, mesh=mesh,
        out_type=jax.ShapeDtypeStruct((M_PAD, ADC), jnp.float32),
        scratch_types=[
            pltpu.VMEM((GB,), jnp.int32),
            pltpu.VMEM((GB, ADC), jnp.float32),
            pltpu.SemaphoreType.DMA,
        ],
    )
    def gather(table_hbm, idx_hbm, out_hbm, idx_v, rows_v, sem):
        wid = lax.axis_index("s") * 2 + lax.axis_index("c")
        base = wid * GB
        pltpu.sync_copy(idx_hbm.at[pl.ds(base, GB)], idx_v)
        pltpu.async_copy(table_hbm.at[idx_v], rows_v, sem).wait()
        pltpu.sync_copy(rows_v, out_hbm.at[pl.ds(base, GB)])

    return gather(ad_table, top_idx)


def _decode8(a0, a1, a2, a3, d0, d1, d2, d3):
    w = a2 - a0
    h = a3 - a1
    cx = a0 + 0.5 * w
    cy = a1 + 0.5 * h
    dw = jnp.clip(d2, -4.0, 4.0)
    dh = jnp.clip(d3, -4.0, 4.0)
    pcx = d0 * w + cx
    pcy = d1 * h + cy
    pw = jnp.exp(dw) * w
    ph = jnp.exp(dh) * h
    x1 = jnp.clip(pcx - 0.5 * pw, 0.0, IMG_W)
    y1 = jnp.clip(pcy - 0.5 * ph, 0.0, IMG_H)
    x2 = jnp.clip(pcx + 0.5 * pw, 0.0, IMG_W)
    y2 = jnp.clip(pcy + 0.5 * ph, 0.0, IMG_H)
    return x1, y1, x2, y2


def _nms_body(ad_ref, adt_ref, out_ref, q_ref, bx_ref):
    x1r, y1r, x2r, y2r = _decode8(*(adt_ref[i:i + 1, :] for i in range(8)))
    area_r = (x2r - x1r) * (y2r - y1r)

    for c in range(N_CHUNKS):
        ad = ad_ref[c * CHUNK:(c + 1) * CHUNK, :]
        x1i, y1i, x2i, y2i = _decode8(*(ad[:, i:i + 1] for i in range(8)))
        bx_ref[c * CHUNK:(c + 1) * CHUNK, :] = jnp.concatenate(
            [x1i, y1i, x2i, y2i], axis=1)
        area_i = (x2i - x1i) * (y2i - y1i)
        if c > 0:
            q_ref[c * CHUNK:(c + 1) * CHUNK, :c * CHUNK] = jnp.zeros(
                (CHUNK, c * CHUNK), jnp.bfloat16)
        sl = slice(c * CHUNK, (c + 1) * CHUNK)
        wx = jnp.clip(jnp.minimum(x2i, x2r[:, sl]) - jnp.maximum(x1i, x1r[:, sl]), 0.0)
        wy = jnp.clip(jnp.minimum(y2i, y2r[:, sl]) - jnp.maximum(y1i, y1r[:, sl]), 0.0)
        inter = wx * wy
        iou = inter / (area_i + area_r[:, sl] - inter + 1e-9)
        al = jax.lax.broadcasted_iota(jnp.int32, (CHUNK, CHUNK), 0)
        bl = jax.lax.broadcasted_iota(jnp.int32, (CHUNK, CHUNK), 1)
        q_ref[sl, sl] = ((iou > NMS_THRESH) & (al < bl)).astype(jnp.bfloat16)
        if c + 1 < N_CHUNKS:
            sr = slice((c + 1) * CHUNK, M_PAD)
            wx = jnp.clip(jnp.minimum(x2i, x2r[:, sr]) - jnp.maximum(x1i, x1r[:, sr]), 0.0)
            wy = jnp.clip(jnp.minimum(y2i, y2r[:, sr]) - jnp.maximum(y1i, y1r[:, sr]), 0.0)
            inter = wx * wy
            iou = inter / (area_i + area_r[:, sr] - inter + 1e-9)
            q_ref[sl, sr] = (iou > NMS_THRESH).astype(jnp.bfloat16)

    valid = (jax.lax.broadcasted_iota(jnp.int32, (1, M_PAD), 1)
             < PRE_NMS_TOPK)
    k0 = valid.astype(jnp.float32)

    def cond(carry):
        return carry[1]

    def body(carry):
        k, _ = carry
        cnt = jnp.dot(k.astype(jnp.bfloat16), q_ref[...],
                      preferred_element_type=jnp.float32)
        k_new = jnp.where((cnt == 0.0) & valid, 1.0, 0.0)
        return k_new, jnp.any(k_new != k)

    k, _ = jax.lax.while_loop(cond, body, (k0, jnp.bool_(True)))

    u128 = (jax.lax.broadcasted_iota(jnp.int32, (CHUNK, CHUNK), 0)
            < jax.lax.broadcasted_iota(jnp.int32, (CHUNK, CHUNK), 1)
            ).astype(jnp.float32)
    rr = jax.lax.broadcasted_iota(jnp.int32, (POST_NMS_TOPK, CHUNK), 0)
    acc = jnp.zeros((POST_NMS_TOPK, 4), jnp.float32)
    base = jnp.float32(0.0)
    for c in range(N_CHUNKS):
        kc = k[:, c * CHUNK:(c + 1) * CHUNK]
        slot = base + jnp.dot(kc, u128, preferred_element_type=jnp.float32)
        base = base + jnp.sum(kc)
        pt = ((slot.astype(jnp.int32) == rr) & (kc == 1.0)).astype(jnp.float32)
        bc = bx_ref[c * CHUNK:(c + 1) * CHUNK, :]
        acc += jnp.dot(pt, bc, preferred_element_type=jnp.float32)
    out_ref[...] = acc


def _nms_compact(ad):
    return pl.pallas_call(
        _nms_body,
        out_shape=jax.ShapeDtypeStruct((POST_NMS_TOPK, 4), jnp.float32),
        scratch_shapes=[pltpu.VMEM((M_PAD, M_PAD), jnp.bfloat16),
                        pltpu.VMEM((M_PAD, 4), jnp.float32)],
        interpret=_INTERPRET,
    )(ad, ad.T)


def kernel(features, conv_w, conv_b, obj_w, obj_b, delta_w, delta_b, anchors):
    scores, deltas = _conv_head(
        features, conv_w, conv_b, obj_w, obj_b, delta_w, delta_b)
    top_scores, top_idx = jax.lax.top_k(scores, PRE_NMS_TOPK)
    ad_table = jnp.concatenate(
        [anchors, deltas, jnp.zeros((N_ANCHORS, ADC - 8), jnp.float32)],
        axis=1)
    idx_pad = jnp.concatenate(
        [top_idx.astype(jnp.int32),
         jnp.zeros((M_PAD - PRE_NMS_TOPK,), jnp.int32)])
    ad = _sc_gather(ad_table, idx_pad)[:, :16]
    return _nms_compact(ad)
